# baseline (speedup 1.0000x reference)
import functools

import jax
import jax.numpy as jnp
from jax import lax
from jax.experimental import pallas as pl
from jax.experimental.pallas import tpu as pltpu
from jax.experimental.pallas import tpu_sc as plsc

NUM_CORES = 2
NUM_SUBCORES = 16
NUM_WORKERS = NUM_CORES * NUM_SUBCORES
BATCH = 16384
EMBED = 32
B_PER_W = BATCH // NUM_WORKERS
CHUNK = 128
NCHUNK = B_PER_W // CHUNK
LANES = 16
TILE = 128
NTILE = B_PER_W // TILE
VPT = TILE // LANES


def _sc_kernel(uid_hbm, iid_hbm, ut_hbm, it_hbm, par_hbm, out_hbm,
               uidx_v, iidx_v, ubf, ibf, urows, irows, par_v, out_v,
               isem, gsem0, gsem1, gsem2, gsem3):
    wid = lax.axis_index("s") * NUM_CORES + lax.axis_index("c")
    base = wid * B_PER_W
    gsems = [gsem0, gsem1, gsem2, gsem3]

    idescs = [pltpu.async_copy(par_hbm, par_v, isem)]
    for c in range(NCHUNK):
        idescs.append(pltpu.async_copy(
            uid_hbm.at[pl.ds(base + c * CHUNK, CHUNK)], uidx_v.at[c], isem))
        idescs.append(pltpu.async_copy(
            iid_hbm.at[pl.ds(base + c * CHUNK, CHUNK)], iidx_v.at[c], isem))
    for dsc in idescs:
        dsc.wait()

    gdescs = []
    for c in range(NCHUNK):
        gdescs.append((
            pltpu.async_copy(ut_hbm.at[uidx_v.at[c]],
                             ubf.at[pl.ds(c * CHUNK, CHUNK)], gsems[c]),
            pltpu.async_copy(it_hbm.at[iidx_v.at[c]],
                             ibf.at[pl.ds(c * CHUNK, CHUNK)], gsems[c]),
        ))

    iota = lax.iota(jnp.int32, LANES)
    b_bc = plsc.load_gather(par_v, [jnp.full((LANES,), 2 * EMBED + 1, jnp.int32)])
    himask = jnp.full((LANES,), -65536, jnp.int32)

    def widen_row(bf_ref, f32_ref, row):
        x = plsc.bitcast(bf_ref[row], jnp.int32)
        ev = plsc.bitcast(lax.shift_left(x, 16), jnp.float32)
        od = plsc.bitcast(lax.bitwise_and(x, himask), jnp.float32)
        f32_ref[row, pl.ds(0, LANES)] = ev
        f32_ref[row, pl.ds(LANES, LANES)] = od

    def tile_body(t, carry):
        gdescs[t][0].wait()
        gdescs[t][1].wait()
        rb = t * TILE

        def widen_body(r, carry2):
            widen_row(ubf, urows, rb + r)
            widen_row(ibf, irows, rb + r)
            return carry2
        lax.fori_loop(0, TILE, widen_body, 0)

        rowidx = [rb + j * LANES + iota for j in range(VPT)]
        acc = [b_bc for _ in range(VPT)]
        for d in range(EMBED):
            cd = jnp.full((LANES,), d, jnp.int32)
            wu = plsc.load_gather(par_v, [jnp.full((LANES,), 1 + d, jnp.int32)])
            wi = plsc.load_gather(par_v, [jnp.full((LANES,), 1 + EMBED + d, jnp.int32)])
            for j in range(VPT):
                gu = plsc.load_gather(urows, [rowidx[j], cd])
                gi = plsc.load_gather(irows, [rowidx[j], cd])
                acc[j] = acc[j] + gu * wu + gi * wi
        for j in range(VPT):
            out_v[pl.ds(rb + j * LANES, LANES)] = acc[j]
        return carry

    for t in range(NTILE):
        tile_body(t, 0)
    pltpu.sync_copy(out_v, out_hbm.at[pl.ds(base, B_PER_W)])


@functools.partial(
    pl.kernel,
    out_type=jax.ShapeDtypeStruct((BATCH,), jnp.float32),
    mesh=plsc.VectorSubcoreMesh(
        core_axis_name="c", subcore_axis_name="s",
        num_cores=NUM_CORES, num_subcores=NUM_SUBCORES),
    compiler_params=pltpu.CompilerParams(
        needs_layout_passes=False, use_tc_tiling_on_sc=False),
    scratch_types=[
        pltpu.VMEM((NCHUNK, CHUNK), jnp.int32),
        pltpu.VMEM((NCHUNK, CHUNK), jnp.int32),
        pltpu.VMEM((B_PER_W, EMBED), jnp.bfloat16),
        pltpu.VMEM((B_PER_W, EMBED), jnp.bfloat16),
        pltpu.VMEM((B_PER_W, EMBED), jnp.float32),
        pltpu.VMEM((B_PER_W, EMBED), jnp.float32),
        pltpu.VMEM((128,), jnp.float32),
        pltpu.VMEM((B_PER_W,), jnp.float32),
        pltpu.SemaphoreType.DMA,
        pltpu.SemaphoreType.DMA,
        pltpu.SemaphoreType.DMA,
        pltpu.SemaphoreType.DMA,
        pltpu.SemaphoreType.DMA,
    ],
)
def _recommender_sc(uid, iid, ut, it, par, out, *scratch):
    _sc_kernel(uid, iid, ut, it, par, out, *scratch)


_PERM = [2 * j for j in range(LANES)] + [2 * j + 1 for j in range(LANES)]


def kernel(user_id, item_id, user_table, item_table, fc_w, fc_b):
    w = fc_w.reshape(2, EMBED)[:, jnp.asarray(_PERM)].reshape(2 * EMBED)
    par = jnp.concatenate([
        jnp.zeros((1,), jnp.float32),
        w.astype(jnp.float32),
        jnp.broadcast_to(fc_b.astype(jnp.float32), (127 - 2 * EMBED,)),
    ])
    out = _recommender_sc(
        user_id.astype(jnp.int32), item_id.astype(jnp.int32),
        user_table.T.astype(jnp.bfloat16).T,
        item_table.T.astype(jnp.bfloat16).T, par)
    return out.reshape(BATCH, 1)


# speedup vs baseline: 1.1822x; 1.1822x over previous
import functools

import jax
import jax.numpy as jnp
from jax import lax
from jax.experimental import pallas as pl
from jax.experimental.pallas import tpu as pltpu
from jax.experimental.pallas import tpu_sc as plsc

NUM_CORES = 2
NUM_SUBCORES = 16
NUM_WORKERS = NUM_CORES * NUM_SUBCORES
BATCH = 16384
EMBED = 32
B_PER_W = BATCH // NUM_WORKERS
CHUNK = 128
NCHUNK = B_PER_W // CHUNK
LANES = 16
TILE = 128
NTILE = B_PER_W // TILE
VPT = TILE // LANES


def _sc_kernel(uid_hbm, iid_hbm, ut_hbm, it_hbm, par_hbm, out_hbm,
               uidx_v, iidx_v, urows, irows, par_v, out_v,
               isem, gsem0, gsem1, gsem2, gsem3):
    wid = lax.axis_index("s") * NUM_CORES + lax.axis_index("c")
    base = wid * B_PER_W
    gsems = [gsem0, gsem1, gsem2, gsem3]

    idescs = [pltpu.async_copy(par_hbm, par_v, isem)]
    for c in range(NCHUNK):
        idescs.append(pltpu.async_copy(
            uid_hbm.at[pl.ds(base + c * CHUNK, CHUNK)], uidx_v.at[c], isem))
        idescs.append(pltpu.async_copy(
            iid_hbm.at[pl.ds(base + c * CHUNK, CHUNK)], iidx_v.at[c], isem))
    for dsc in idescs:
        dsc.wait()

    gdescs = []
    for c in range(NCHUNK):
        gdescs.append((
            pltpu.async_copy(ut_hbm.at[uidx_v.at[c]],
                             urows.at[pl.ds(c * CHUNK, CHUNK)], gsems[c]),
            pltpu.async_copy(it_hbm.at[iidx_v.at[c]],
                             irows.at[pl.ds(c * CHUNK, CHUNK)], gsems[c]),
        ))

    iota = lax.iota(jnp.int32, LANES)
    b_bc = plsc.load_gather(par_v, [jnp.full((LANES,), 2 * EMBED + 1, jnp.int32)])

    def tile_body(t, carry):
        gdescs[t][0].wait()
        gdescs[t][1].wait()
        rb = t * TILE
        rowidx = [rb + j * LANES + iota for j in range(VPT)]
        acc = [b_bc for _ in range(VPT)]
        for d in range(EMBED):
            cd = jnp.full((LANES,), d, jnp.int32)
            wu = plsc.load_gather(par_v, [jnp.full((LANES,), 1 + d, jnp.int32)])
            wi = plsc.load_gather(par_v, [jnp.full((LANES,), 1 + EMBED + d, jnp.int32)])
            for j in range(VPT):
                gu = plsc.load_gather(urows, [rowidx[j], cd])
                gi = plsc.load_gather(irows, [rowidx[j], cd])
                acc[j] = acc[j] + gu * wu + gi * wi
        for j in range(VPT):
            out_v[pl.ds(rb + j * LANES, LANES)] = acc[j]
        return carry

    for t in range(NTILE):
        tile_body(t, 0)
    pltpu.sync_copy(out_v, out_hbm.at[pl.ds(base, B_PER_W)])


@functools.partial(
    pl.kernel,
    out_type=jax.ShapeDtypeStruct((BATCH,), jnp.float32),
    mesh=plsc.VectorSubcoreMesh(
        core_axis_name="c", subcore_axis_name="s",
        num_cores=NUM_CORES, num_subcores=NUM_SUBCORES),
    compiler_params=pltpu.CompilerParams(
        needs_layout_passes=False, use_tc_tiling_on_sc=False),
    scratch_types=[
        pltpu.VMEM((NCHUNK, CHUNK), jnp.int32),
        pltpu.VMEM((NCHUNK, CHUNK), jnp.int32),
        pltpu.VMEM((B_PER_W, EMBED), jnp.float32),
        pltpu.VMEM((B_PER_W, EMBED), jnp.float32),
        pltpu.VMEM((128,), jnp.float32),
        pltpu.VMEM((B_PER_W,), jnp.float32),
        pltpu.SemaphoreType.DMA,
        pltpu.SemaphoreType.DMA,
        pltpu.SemaphoreType.DMA,
        pltpu.SemaphoreType.DMA,
        pltpu.SemaphoreType.DMA,
    ],
)
def _recommender_sc(uid, iid, ut, it, par, out, *scratch):
    _sc_kernel(uid, iid, ut, it, par, out, *scratch)


def kernel(user_id, item_id, user_table, item_table, fc_w, fc_b):
    par = jnp.concatenate([
        jnp.zeros((1,), jnp.float32),
        fc_w.reshape(2 * EMBED).astype(jnp.float32),
        jnp.broadcast_to(fc_b.astype(jnp.float32), (127 - 2 * EMBED,)),
    ])
    out = _recommender_sc(
        user_id.astype(jnp.int32), item_id.astype(jnp.int32),
        user_table, item_table, par)
    return out.reshape(BATCH, 1)
